# baseline (speedup 1.0000x reference)
import jax
import jax.numpy as jnp
from jax.experimental import pallas as pl

_TOKENS = 16384
_PATHS = 64
_ROWS = (_TOKENS * _PATHS) // 128
_BLOCK_ROWS = 2048


def _body(s_ref, o_ref):
    s = s_ref[...]
    pos = s > 0.0
    col = jax.lax.broadcasted_iota(jnp.int32, s.shape, 1)
    lmax = jnp.max(s[:, :64], axis=1, keepdims=True)
    rmax = jnp.max(s[:, 64:], axis=1, keepdims=True)
    lresid = (col == 0) & (lmax <= 0.0)
    rresid = (col == 64) & (rmax <= 0.0)
    o_ref[...] = jnp.where(pos | lresid | rresid, 1, 0).astype(jnp.int32)


def kernel(score):
    s2 = score.reshape(_ROWS, 128)
    out = pl.pallas_call(
        _body,
        out_shape=jax.ShapeDtypeStruct((_ROWS, 128), jnp.int32),
        grid=(_ROWS // _BLOCK_ROWS,),
        in_specs=[pl.BlockSpec((_BLOCK_ROWS, 128), lambda i: (i, 0))],
        out_specs=pl.BlockSpec((_BLOCK_ROWS, 128), lambda i: (i, 0)),
    )(s2)
    return out.reshape(_TOKENS, _PATHS)


# speedup vs baseline: 1.6470x; 1.6470x over previous
import jax
import jax.numpy as jnp
from jax.experimental import pallas as pl

_TOKENS = 16384
_PATHS = 64
_BLOCK_ROWS = 4096


def _body(s_ref, o_ref):
    s = s_ref[...]
    pos = s > 0.0
    col = jax.lax.broadcasted_iota(jnp.int32, s.shape, 1)
    rmax = jnp.max(s, axis=1, keepdims=True)
    resid = (col == 0) & (rmax <= 0.0)
    o_ref[...] = jnp.where(pos | resid, 1, 0).astype(jnp.int32)


def kernel(score):
    return pl.pallas_call(
        _body,
        out_shape=jax.ShapeDtypeStruct((_TOKENS, _PATHS), jnp.int32),
        grid=(_TOKENS // _BLOCK_ROWS,),
        in_specs=[pl.BlockSpec((_BLOCK_ROWS, _PATHS), lambda i: (i, 0))],
        out_specs=pl.BlockSpec((_BLOCK_ROWS, _PATHS), lambda i: (i, 0)),
    )(score)
